# baseline (speedup 1.0000x reference)
import math

import jax
import jax.numpy as jnp
from jax.experimental import pallas as pl
from jax.experimental.pallas import tpu as pltpu

FEATS = 64
NPF = FEATS // 2
H = 64
W = 64
HW = H * W
B = 4
NUM_OFFSETS = 7
_TEMPERATURE = 10000.0
_SCALE = 2.0 * math.pi
_EPS = 1e-6
_NCHUNK = 8
_RC = FEATS // _NCHUNK


def _pe_table():
    c = jnp.arange(FEATS)[:, None]
    col = jnp.arange(HW)[None, :]
    y = col // W
    x = col - y * W
    is_y = c < NPF
    cm = jnp.where(is_y, c, c - NPF)
    k = cm // 2
    e = (jnp.where(is_y, y, x).astype(jnp.float32) + 1.0) * (
        _SCALE / (float(H) + _EPS)
    )
    inv_d = jnp.exp(k.astype(jnp.float32) * (-math.log(_TEMPERATURE) * 2.0 / NPF))
    arg = e * inv_d
    return jnp.where(cm % 2 == 0, jnp.sin(arg), jnp.cos(arg))


def _body(pos_ref, offs_ref, pe_hbm, out_ref, pe_v, insems, outsems):
    loads = [
        pltpu.make_async_copy(
            pe_hbm.at[pl.ds(j * _RC, _RC)], pe_v.at[pl.ds(j * _RC, _RC)], insems.at[j]
        )
        for j in range(_NCHUNK)
    ]
    for ld in loads:
        ld.start()
    pos = pos_ref[0, 0]
    row = jax.lax.broadcasted_iota(jnp.int32, (NUM_OFFSETS, FEATS), 0)
    off_row = jnp.sum(
        jnp.where(row == pos, offs_ref[...], 0.0), axis=0, keepdims=True
    )
    ci = jax.lax.broadcasted_iota(jnp.int32, (FEATS, FEATS), 1)
    ri = jax.lax.broadcasted_iota(jnp.int32, (FEATS, FEATS), 0)
    off = jnp.sum(
        jnp.where(ci == ri, jnp.broadcast_to(off_row, (FEATS, FEATS)), 0.0),
        axis=1,
        keepdims=True,
    )

    copies = []
    for j in range(_NCHUNK):
        c0 = j * _RC
        loads[j].wait()
        pe_v[pl.ds(c0, _RC)] = pe_v[pl.ds(c0, _RC)] + off[c0 : c0 + _RC]
        for b in range(B):
            cp = pltpu.make_async_copy(
                pe_v.at[pl.ds(c0, _RC)], out_ref.at[b, pl.ds(c0, _RC)], outsems.at[b]
            )
            cp.start()
            copies.append(cp)
    for cp in copies:
        cp.wait()


def kernel(base_memposenc_offsets, imagelike_shape_bchw, position_offset):
    del imagelike_shape_bchw
    offs = base_memposenc_offsets.reshape(NUM_OFFSETS, FEATS)
    pos = jnp.asarray(position_offset, jnp.int32).reshape(1, 1)
    out = pl.pallas_call(
        _body,
        in_specs=[
            pl.BlockSpec(memory_space=pltpu.SMEM),
            pl.BlockSpec(memory_space=pltpu.VMEM),
            pl.BlockSpec(memory_space=pl.ANY),
        ],
        out_specs=pl.BlockSpec(memory_space=pl.ANY),
        out_shape=jax.ShapeDtypeStruct((B, FEATS, HW), jnp.float32),
        scratch_shapes=[
            pltpu.VMEM((FEATS, HW), jnp.float32),
            pltpu.SemaphoreType.DMA((_NCHUNK,)),
            pltpu.SemaphoreType.DMA((B,)),
        ],
    )(pos, offs, _pe_table())
    return out.reshape(B, FEATS, H, W)


# speedup vs baseline: 1.2303x; 1.2303x over previous
import math

import jax
import jax.numpy as jnp
from jax.experimental import pallas as pl
from jax.experimental.pallas import tpu as pltpu

FEATS = 64
NPF = FEATS // 2
H = 64
W = 64
HW = H * W
B = 4
NUM_OFFSETS = 7
_TEMPERATURE = 10000.0
_SCALE = 2.0 * math.pi
_EPS = 1e-6
_NCHUNK = 8
_RC = FEATS // _NCHUNK


def _body(pos_ref, offs_ref, out_ref, sel_ref, sems):
    pos = pos_ref[0, 0]
    row = jax.lax.broadcasted_iota(jnp.int32, (NUM_OFFSETS, FEATS), 0)
    off_row = jnp.sum(
        jnp.where(row == pos, offs_ref[...], 0.0), axis=0, keepdims=True
    )
    ci = jax.lax.broadcasted_iota(jnp.int32, (FEATS, FEATS), 1)
    ri = jax.lax.broadcasted_iota(jnp.int32, (FEATS, FEATS), 0)
    off = jnp.sum(
        jnp.where(ci == ri, jnp.broadcast_to(off_row, (FEATS, FEATS)), 0.0),
        axis=1,
        keepdims=True,
    )

    col = jax.lax.broadcasted_iota(jnp.int32, (_RC, HW), 1)
    crow = jax.lax.broadcasted_iota(jnp.int32, (_RC, 1), 0)

    copies = []
    e = None
    for j in range(_NCHUNK):
        c0 = j * _RC
        if j == 0:
            e = ((col // W).astype(jnp.float32) + 1.0) * (
                _SCALE / (float(H) + _EPS)
            )
        elif c0 == NPF:
            e = ((col % W).astype(jnp.float32) + 1.0) * (
                _SCALE / (float(W) + _EPS)
            )
        cm = crow + (c0 if c0 < NPF else c0 - NPF)
        k = cm // 2
        inv_d = jnp.exp(
            k.astype(jnp.float32) * (-math.log(_TEMPERATURE) * 2.0 / NPF)
        )
        phase = (cm % 2).astype(jnp.float32) * (0.5 * math.pi)
        sel_ref[pl.ds(c0, _RC)] = jnp.sin(e * inv_d + phase) + off[c0 : c0 + _RC]
        for b in range(B):
            cp = pltpu.make_async_copy(
                sel_ref.at[pl.ds(c0, _RC)], out_ref.at[b, pl.ds(c0, _RC)], sems.at[b]
            )
            cp.start()
            copies.append(cp)
    for cp in copies:
        cp.wait()


def kernel(base_memposenc_offsets, imagelike_shape_bchw, position_offset):
    del imagelike_shape_bchw
    offs = base_memposenc_offsets.reshape(NUM_OFFSETS, FEATS)
    pos = jnp.asarray(position_offset, jnp.int32).reshape(1, 1)
    out = pl.pallas_call(
        _body,
        in_specs=[
            pl.BlockSpec(memory_space=pltpu.SMEM),
            pl.BlockSpec(memory_space=pltpu.VMEM),
        ],
        out_specs=pl.BlockSpec(memory_space=pl.ANY),
        out_shape=jax.ShapeDtypeStruct((B, FEATS, HW), jnp.float32),
        scratch_shapes=[
            pltpu.VMEM((FEATS, HW), jnp.float32),
            pltpu.SemaphoreType.DMA((B,)),
        ],
    )(pos, offs)
    return out.reshape(B, FEATS, H, W)


# speedup vs baseline: 1.4189x; 1.1533x over previous
import math

import jax
import jax.numpy as jnp
from jax.experimental import pallas as pl
from jax.experimental.pallas import tpu as pltpu

FEATS = 64
NPF = FEATS // 2
H = 64
W = 64
HW = H * W
B = 4
NUM_OFFSETS = 7
_TEMPERATURE = 10000.0
_SCALE = 2.0 * math.pi
_EPS = 1e-6
_NCHUNK = 8
_RC = FEATS // _NCHUNK


def _body(pos_ref, offs_ref, out_ref, sel_ref, sems):
    pos = pos_ref[0, 0]
    row = jax.lax.broadcasted_iota(jnp.int32, (NUM_OFFSETS, FEATS), 0)
    off_row = jnp.sum(
        jnp.where(row == pos, offs_ref[...], 0.0), axis=0, keepdims=True
    )
    ci = jax.lax.broadcasted_iota(jnp.int32, (FEATS, FEATS), 1)
    ri = jax.lax.broadcasted_iota(jnp.int32, (FEATS, FEATS), 0)
    off = jnp.sum(
        jnp.where(ci == ri, jnp.broadcast_to(off_row, (FEATS, FEATS)), 0.0),
        axis=1,
        keepdims=True,
    )

    xv = jax.lax.broadcasted_iota(jnp.int32, (_RC, W), 1)
    crow = jax.lax.broadcasted_iota(jnp.int32, (_RC, 1), 0)
    e = (xv.astype(jnp.float32) + 1.0) * (_SCALE / (float(W) + _EPS))

    copies = []

    def _emit(c0):
        for b in range(B):
            cp = pltpu.make_async_copy(
                sel_ref.at[pl.ds(c0, _RC)], out_ref.at[b, pl.ds(c0, _RC)], sems.at[b]
            )
            cp.start()
            copies.append(cp)

    for j in range(_NCHUNK // 2):
        c0 = j * _RC
        cm = crow + c0
        k = cm // 2
        inv_d = jnp.exp(
            k.astype(jnp.float32) * (-math.log(_TEMPERATURE) * 2.0 / NPF)
        )
        phase = (cm % 2).astype(jnp.float32) * (0.5 * math.pi)
        s_base = jnp.sin(e * inv_d + phase)
        y_rows = s_base + off[c0 : c0 + _RC]
        sel_ref[pl.ds(c0, _RC)] = jax.lax.transpose(
            jnp.broadcast_to(y_rows[:, None, :], (_RC, H, W)), (0, 2, 1)
        ).reshape(_RC, HW)
        _emit(c0)
        x_rows = s_base + off[NPF + c0 : NPF + c0 + _RC]
        sel_ref[pl.ds(NPF + c0, _RC)] = jnp.broadcast_to(
            x_rows[:, None, :], (_RC, H, W)
        ).reshape(_RC, HW)
        _emit(NPF + c0)
    for cp in copies:
        cp.wait()


def kernel(base_memposenc_offsets, imagelike_shape_bchw, position_offset):
    del imagelike_shape_bchw
    offs = base_memposenc_offsets.reshape(NUM_OFFSETS, FEATS)
    pos = jnp.asarray(position_offset, jnp.int32).reshape(1, 1)
    out = pl.pallas_call(
        _body,
        in_specs=[
            pl.BlockSpec(memory_space=pltpu.SMEM),
            pl.BlockSpec(memory_space=pltpu.VMEM),
        ],
        out_specs=pl.BlockSpec(memory_space=pl.ANY),
        out_shape=jax.ShapeDtypeStruct((B, FEATS, HW), jnp.float32),
        scratch_shapes=[
            pltpu.VMEM((FEATS, HW), jnp.float32),
            pltpu.SemaphoreType.DMA((B,)),
        ],
    )(pos, offs)
    return out.reshape(B, FEATS, H, W)
